# Initial kernel scaffold; baseline (speedup 1.0000x reference)
import functools

import jax
import jax.numpy as jnp
from jax import lax
from jax.experimental import pallas as pl
from jax.experimental.pallas import tpu as pltpu
from jax.experimental.pallas import tpu_sc as plsc

_SP = (96, 96, 48)
_DSP = (48, 48, 24)
_NPT = 50000
_NPAD = 50176
_NW = 32
_NC = 2
_BPW = _NPAD // _NW
_BN = 512
_NBLK = _NPAD // _BN
_EPS = 1e-5


def _enc(c, shape):
    return (c[..., 0] * shape[1] + c[..., 1]) * shape[2] + c[..., 2]


def _build_maps(coords):
    M = _SP[0] * _SP[1] * _SP[2]
    Md = _DSP[0] * _DSP[1] * _DSP[2]
    sh = jnp.array(_SP, jnp.int32)
    dsh = jnp.array(_DSP, jnp.int32)

    keys = _enc(coords, _SP)
    ftab = jnp.full((M,), _NPT, jnp.int32).at[keys].set(
        jnp.arange(_NPT, dtype=jnp.int32))

    subm1 = []
    for dz in (-1, 0, 1):
        for dy in (-1, 0, 1):
            for dx in (-1, 0, 1):
                nbr = coords + jnp.array([dz, dy, dx], jnp.int32)
                valid = jnp.all((nbr >= 0) & (nbr < sh), axis=1)
                q = _enc(jnp.clip(nbr, 0, sh - 1), _SP)
                subm1.append(jnp.where(valid, ftab[q], _NPT))
    subm1 = jnp.concatenate(
        [jnp.stack(subm1),
         jnp.full((27, _NPAD - _NPT), _NPT, jnp.int32)], axis=1)

    ck = _enc(coords // 2, _DSP)
    occ = jnp.zeros((Md,), jnp.int32).at[ck].set(1)
    ranks = jnp.cumsum(occ) - occ
    cnt = jnp.sum(occ)
    ctab = jnp.where(occ == 1, ranks, _NPAD - 1)
    rowkey = jnp.full((_NPAD,), Md, jnp.int32).at[
        jnp.where(occ == 1, ranks, _NPAD)].set(
        jnp.arange(Md, dtype=jnp.int32), mode="drop")
    ox = rowkey % _DSP[2]
    oy = (rowkey // _DSP[2]) % _DSP[1]
    oz = rowkey // (_DSP[2] * _DSP[1])
    oc = jnp.stack([oz, oy, ox], axis=1).astype(jnp.int32)
    rvalid = rowkey < Md

    subm2 = []
    for dz in (-1, 0, 1):
        for dy in (-1, 0, 1):
            for dx in (-1, 0, 1):
                nbr = oc + jnp.array([dz, dy, dx], jnp.int32)
                valid = rvalid & jnp.all((nbr >= 0) & (nbr < dsh), axis=1)
                q = _enc(jnp.clip(nbr, 0, dsh - 1), _DSP)
                subm2.append(jnp.where(valid, ctab[q], _NPAD - 1))
    subm2 = jnp.stack(subm2)

    down = []
    for dz in (0, 1):
        for dy in (0, 1):
            for dx in (0, 1):
                nbr = oc * 2 + jnp.array([dz, dy, dx], jnp.int32)
                valid = rvalid & jnp.all(nbr < sh, axis=1)
                q = _enc(jnp.clip(nbr, 0, sh - 1), _SP)
                down.append(jnp.where(valid, ftab[q], _NPT))
    down = jnp.stack(down)

    inv_row = jnp.concatenate(
        [ctab[ck], jnp.full((_NPAD - _NPT,), _NPAD - 1, jnp.int32)])[None, :]
    rem = coords % 2
    invk = (rem[:, 0] * 2 + rem[:, 1]) * 2 + rem[:, 2]
    oh = (invk[:, None] == jnp.arange(8, dtype=jnp.int32)[None, :])
    oh = jnp.concatenate(
        [oh.astype(jnp.float32), jnp.zeros((_NPAD - _NPT, 8), jnp.float32)],
        axis=0)
    return subm1, subm2, down, inv_row, oh, cnt.astype(jnp.float32)


def _sc_gather(table, idx):
    K = idx.shape[0]
    C = table.shape[1]
    mesh = plsc.VectorSubcoreMesh(core_axis_name="c", subcore_axis_name="s")

    @functools.partial(
        pl.kernel, mesh=mesh,
        out_type=jax.ShapeDtypeStruct((K, _NPAD, C), jnp.float32),
        scratch_types=[
            pltpu.VMEM((_BPW,), jnp.int32),
            pltpu.VMEM((_BPW, C), jnp.float32),
            pltpu.SemaphoreType.DMA,
        ],
    )
    def gk(table_hbm, idx_hbm, out_hbm, idx_v, rows_v, sem):
        wid = lax.axis_index("s") * _NC + lax.axis_index("c")
        base = wid * _BPW
        for k in range(K):
            pltpu.sync_copy(idx_hbm.at[k, pl.ds(base, _BPW)], idx_v)
            pltpu.async_copy(table_hbm.at[idx_v], rows_v, sem).wait()
            pltpu.sync_copy(rows_v, out_hbm.at[k, pl.ds(base, _BPW)])

    return gk(table, idx)


def _conv_body(g_ref, w_ref, y_ref, s_ref, *, taps):
    acc = jnp.zeros(y_ref.shape, jnp.float32)
    for k in range(taps):
        acc = acc + jnp.dot(g_ref[k], w_ref[k],
                            preferred_element_type=jnp.float32)
    y_ref[...] = acc
    if s_ref is not None:
        @pl.when(pl.program_id(0) == 0)
        def _():
            s_ref[...] = jnp.zeros_like(s_ref)
        ps = jnp.zeros_like(s_ref)
        ps = ps.at[0].set(jnp.sum(acc, axis=0))
        ps = ps.at[1].set(jnp.sum(acc * acc, axis=0))
        s_ref[...] += ps


def _conv_call(G, W, stats):
    K, _, C = G.shape
    Co = W.shape[2]
    outs = [jax.ShapeDtypeStruct((_NPAD, Co), jnp.float32)]
    out_specs = [pl.BlockSpec((_BN, Co), lambda i: (i, 0))]
    if stats:
        body = functools.partial(_conv_body, taps=K)
        outs.append(jax.ShapeDtypeStruct((8, Co), jnp.float32))
        out_specs.append(pl.BlockSpec((8, Co), lambda i: (0, 0)))
    else:
        def body(g_ref, w_ref, y_ref, taps=K):
            _conv_body(g_ref, w_ref, y_ref, None, taps=taps)
    return pl.pallas_call(
        body, grid=(_NBLK,),
        in_specs=[pl.BlockSpec((K, _BN, C), lambda i: (0, i, 0)),
                  pl.BlockSpec((K, C, Co), lambda i: (0, 0, 0))],
        out_specs=out_specs,
        out_shape=outs,
    )(G, W)


def _affine(s_ref, p_ref):
    cntf = p_ref[2, 0]
    m = s_ref[0] / cntf
    v = s_ref[1] / cntf - m * m
    scale = p_ref[0] * lax.rsqrt(v + _EPS)
    shift = p_ref[1] - m * scale
    return scale, shift, cntf


def _norm_body(y_ref, s_ref, p_ref, o_ref):
    scale, shift, cntf = _affine(s_ref, p_ref)
    y = y_ref[...]
    act = jnp.maximum(y * scale[None, :] + shift[None, :], 0.0)
    rows = (lax.broadcasted_iota(jnp.int32, y.shape, 0)
            + pl.program_id(0) * y.shape[0])
    o_ref[...] = jnp.where(rows < cntf.astype(jnp.int32), act, 0.0)


def _norm_call(Y, S, P):
    C = Y.shape[1]
    return pl.pallas_call(
        _norm_body, grid=(_NBLK,),
        in_specs=[pl.BlockSpec((_BN, C), lambda i: (i, 0)),
                  pl.BlockSpec((8, C), lambda i: (0, 0)),
                  pl.BlockSpec((8, C), lambda i: (0, 0))],
        out_specs=pl.BlockSpec((_BN, C), lambda i: (i, 0)),
        out_shape=jax.ShapeDtypeStruct((_NPAD, C), jnp.float32),
    )(Y, S, P)


def _inv_body(g_ref, w_ref, oh_ref, o_ref):
    acc = jnp.zeros(o_ref.shape, jnp.float32)
    g = g_ref[...]
    ohb = oh_ref[...]
    for k in range(8):
        acc = acc + ohb[:, k:k + 1] * jnp.dot(
            g, w_ref[k], preferred_element_type=jnp.float32)
    o_ref[...] = acc


def _inv_call(Ginv, Wu, oh):
    return pl.pallas_call(
        _inv_body, grid=(_NBLK,),
        in_specs=[pl.BlockSpec((_BN, 32), lambda i: (i, 0)),
                  pl.BlockSpec((8, 32, 16), lambda i: (0, 0, 0)),
                  pl.BlockSpec((_BN, 8), lambda i: (i, 0))],
        out_specs=pl.BlockSpec((_BN, 16), lambda i: (i, 0)),
        out_shape=jax.ShapeDtypeStruct((_NPAD, 16), jnp.float32),
    )(Ginv, Wu, oh)


def _final_body(y_ref, s_ref, p_ref, wo_ref, bo_ref, o_ref):
    scale, shift, cntf = _affine(s_ref, p_ref)
    y = y_ref[...]
    act = jnp.maximum(y * scale[None, :] + shift[None, :], 0.0)
    rows = (lax.broadcasted_iota(jnp.int32, y.shape, 0)
            + pl.program_id(0) * y.shape[0])
    act = jnp.where(rows < cntf.astype(jnp.int32), act, 0.0)
    o_ref[...] = (jnp.dot(act, wo_ref[...], preferred_element_type=jnp.float32)
                  + bo_ref[0:1, :])


def _final_call(Y, S, P, Wo, bo):
    bo8 = jnp.broadcast_to(bo[None, :], (8, 8))
    return pl.pallas_call(
        _final_body, grid=(_NBLK,),
        in_specs=[pl.BlockSpec((_BN, 16), lambda i: (i, 0)),
                  pl.BlockSpec((8, 16), lambda i: (0, 0)),
                  pl.BlockSpec((8, 16), lambda i: (0, 0)),
                  pl.BlockSpec((16, 8), lambda i: (0, 0)),
                  pl.BlockSpec((8, 8), lambda i: (0, 0))],
        out_specs=pl.BlockSpec((_BN, 8), lambda i: (i, 0)),
        out_shape=jax.ShapeDtypeStruct((_NPAD, 8), jnp.float32),
    )(Y, S, P, Wo, bo8)


def _params(g, b, cntf):
    C = g.shape[0]
    p = jnp.zeros((8, C), jnp.float32)
    return p.at[0].set(g).at[1].set(b).at[2].set(cntf)


def kernel(feats, coords, W1a, g1a, b1a, W1b, g1b, b1b, Wd, W2a, g2a, b2a,
           W2b, g2b, b2b, Wu, W3a, g3a, b3a, W3b, g3b, b3b, Wo, bo):
    subm1, subm2, down, inv_row, oh, cntf = _build_maps(coords)
    nf = jnp.float32(_NPT)

    feats16 = jnp.zeros((_NPAD, 16), jnp.float32).at[:_NPT, :2].set(feats)
    W1a_p = jnp.zeros((27, 16, 16), jnp.float32).at[:, :2, :].set(W1a)

    Y, S = _conv_call(_sc_gather(feats16, subm1), W1a_p, True)
    act1 = _norm_call(Y, S, _params(g1a, b1a, nf))

    Y, S = _conv_call(_sc_gather(act1, subm1), W1b, True)
    skip1 = _norm_call(Y, S, _params(g1b, b1b, nf))

    (xd,) = _conv_call(_sc_gather(skip1, down), Wd, False)

    Y, S = _conv_call(_sc_gather(xd, subm2), W2a, True)
    act2a = _norm_call(Y, S, _params(g2a, b2a, cntf))

    Y, S = _conv_call(_sc_gather(act2a, subm2), W2b, True)
    act2b = _norm_call(Y, S, _params(g2b, b2b, cntf))

    up = _inv_call(_sc_gather(act2b, inv_row)[0], Wu, oh)
    cat = jnp.concatenate([up, skip1], axis=1)

    Y, S = _conv_call(_sc_gather(cat, subm1), W3a, True)
    act3a = _norm_call(Y, S, _params(g3a, b3a, nf))

    Y, S = _conv_call(_sc_gather(act3a, subm1), W3b, True)
    return _final_call(Y, S, _params(g3b, b3b, nf), Wo, bo)[:_NPT]


# speedup vs baseline: 1.9592x; 1.9592x over previous
import functools

import jax
import jax.numpy as jnp
from jax import lax
from jax.experimental import pallas as pl
from jax.experimental.pallas import tpu as pltpu
from jax.experimental.pallas import tpu_sc as plsc

_SP = (96, 96, 48)
_DSP = (48, 48, 24)
_NPT = 50000
_NPAD = 50176
_NW = 32
_NC = 2
_BPW = _NPAD // _NW
_BN = 512
_NBLK = _NPAD // _BN
_EPS = 1e-5


def _enc(c, shape):
    return (c[..., 0] * shape[1] + c[..., 1]) * shape[2] + c[..., 2]


def _build_maps(coords):
    M = _SP[0] * _SP[1] * _SP[2]
    Md = _DSP[0] * _DSP[1] * _DSP[2]
    sh = jnp.array(_SP, jnp.int32)
    dsh = jnp.array(_DSP, jnp.int32)

    keys = _enc(coords, _SP)
    ftab = jnp.full((M,), _NPT, jnp.int32).at[keys].set(
        jnp.arange(_NPT, dtype=jnp.int32))

    subm1 = []
    for dz in (-1, 0, 1):
        for dy in (-1, 0, 1):
            for dx in (-1, 0, 1):
                nbr = coords + jnp.array([dz, dy, dx], jnp.int32)
                valid = jnp.all((nbr >= 0) & (nbr < sh), axis=1)
                q = _enc(jnp.clip(nbr, 0, sh - 1), _SP)
                subm1.append(jnp.where(valid, ftab[q], _NPT))
    subm1 = jnp.concatenate(
        [jnp.stack(subm1),
         jnp.full((27, _NPAD - _NPT), _NPT, jnp.int32)], axis=1)

    ck = _enc(coords // 2, _DSP)
    occ = jnp.zeros((Md,), jnp.int32).at[ck].set(1)
    ranks = jnp.cumsum(occ) - occ
    cnt = jnp.sum(occ)
    ctab = jnp.where(occ == 1, ranks, _NPAD - 1)
    rowkey = jnp.full((_NPAD,), Md, jnp.int32).at[
        jnp.where(occ == 1, ranks, _NPAD)].set(
        jnp.arange(Md, dtype=jnp.int32), mode="drop")
    ox = rowkey % _DSP[2]
    oy = (rowkey // _DSP[2]) % _DSP[1]
    oz = rowkey // (_DSP[2] * _DSP[1])
    oc = jnp.stack([oz, oy, ox], axis=1).astype(jnp.int32)
    rvalid = rowkey < Md

    subm2 = []
    for dz in (-1, 0, 1):
        for dy in (-1, 0, 1):
            for dx in (-1, 0, 1):
                nbr = oc + jnp.array([dz, dy, dx], jnp.int32)
                valid = rvalid & jnp.all((nbr >= 0) & (nbr < dsh), axis=1)
                q = _enc(jnp.clip(nbr, 0, dsh - 1), _DSP)
                subm2.append(jnp.where(valid, ctab[q], _NPAD - 1))
    subm2 = jnp.stack(subm2)

    down = []
    for dz in (0, 1):
        for dy in (0, 1):
            for dx in (0, 1):
                nbr = oc * 2 + jnp.array([dz, dy, dx], jnp.int32)
                valid = rvalid & jnp.all(nbr < sh, axis=1)
                q = _enc(jnp.clip(nbr, 0, sh - 1), _SP)
                down.append(jnp.where(valid, ftab[q], _NPT))
    down = jnp.stack(down)

    inv_row = jnp.concatenate(
        [ctab[ck], jnp.full((_NPAD - _NPT,), _NPAD - 1, jnp.int32)])[None, :]
    rem = coords % 2
    invk = (rem[:, 0] * 2 + rem[:, 1]) * 2 + rem[:, 2]
    oh = (invk[:, None] == jnp.arange(8, dtype=jnp.int32)[None, :])
    oh = jnp.concatenate(
        [oh.astype(jnp.float32), jnp.zeros((_NPAD - _NPT, 8), jnp.float32)],
        axis=0)
    return subm1, subm2, down, inv_row, oh, cnt.astype(jnp.float32)


def _sc_gather(table, idx):
    K = idx.shape[0]
    C = table.shape[1]
    idx = idx.reshape((K * _NPAD,))
    mesh = plsc.VectorSubcoreMesh(core_axis_name="c", subcore_axis_name="s")

    @functools.partial(
        pl.kernel, mesh=mesh,
        compiler_params=pltpu.CompilerParams(use_tc_tiling_on_sc=False),
        out_type=jax.ShapeDtypeStruct((K, _NPAD, C), jnp.float32),
        scratch_types=[
            pltpu.VMEM((_BPW,), jnp.int32),
            pltpu.VMEM((_BPW, C), jnp.float32),
            pltpu.SemaphoreType.DMA,
        ],
    )
    def gk(table_hbm, idx_hbm, out_hbm, idx_v, rows_v, sem):
        wid = lax.axis_index("s") * _NC + lax.axis_index("c")
        base = wid * _BPW
        for k in range(K):
            pltpu.sync_copy(idx_hbm.at[pl.ds(k * _NPAD + base, _BPW)], idx_v)
            pltpu.async_copy(table_hbm.at[idx_v], rows_v, sem).wait()
            pltpu.sync_copy(rows_v, out_hbm.at[k, pl.ds(base, _BPW)])

    return gk(table, idx)


def _conv_body(g_ref, w_ref, y_ref, s_ref, *, taps):
    acc = jnp.zeros(y_ref.shape, jnp.float32)
    for k in range(taps):
        acc = acc + jnp.dot(g_ref[k], w_ref[k],
                            preferred_element_type=jnp.float32)
    y_ref[...] = acc
    if s_ref is not None:
        @pl.when(pl.program_id(0) == 0)
        def _():
            s_ref[...] = jnp.zeros_like(s_ref)
        ps = jnp.concatenate(
            [jnp.sum(acc, axis=0)[None, :],
             jnp.sum(acc * acc, axis=0)[None, :],
             jnp.zeros((6, acc.shape[1]), jnp.float32)], axis=0)
        s_ref[...] += ps


def _conv_call(G, W, stats):
    K, _, C = G.shape
    Co = W.shape[2]
    outs = [jax.ShapeDtypeStruct((_NPAD, Co), jnp.float32)]
    out_specs = [pl.BlockSpec((_BN, Co), lambda i: (i, 0))]
    if stats:
        body = functools.partial(_conv_body, taps=K)
        outs.append(jax.ShapeDtypeStruct((8, Co), jnp.float32))
        out_specs.append(pl.BlockSpec((8, Co), lambda i: (0, 0)))
    else:
        def body(g_ref, w_ref, y_ref, taps=K):
            _conv_body(g_ref, w_ref, y_ref, None, taps=taps)
    return pl.pallas_call(
        body, grid=(_NBLK,),
        in_specs=[pl.BlockSpec((K, _BN, C), lambda i: (0, i, 0)),
                  pl.BlockSpec((K, C, Co), lambda i: (0, 0, 0))],
        out_specs=out_specs,
        out_shape=outs,
    )(G, W)


def _affine(s_ref, p_ref):
    cntf = p_ref[2, 0]
    m = s_ref[0] / cntf
    v = s_ref[1] / cntf - m * m
    scale = p_ref[0] * lax.rsqrt(v + _EPS)
    shift = p_ref[1] - m * scale
    return scale, shift, cntf


def _norm_body(y_ref, s_ref, p_ref, o_ref):
    scale, shift, cntf = _affine(s_ref, p_ref)
    y = y_ref[...]
    act = jnp.maximum(y * scale[None, :] + shift[None, :], 0.0)
    rows = (lax.broadcasted_iota(jnp.int32, y.shape, 0)
            + pl.program_id(0) * y.shape[0])
    o_ref[...] = jnp.where(rows < cntf.astype(jnp.int32), act, 0.0)


def _norm_call(Y, S, P):
    C = Y.shape[1]
    return pl.pallas_call(
        _norm_body, grid=(_NBLK,),
        in_specs=[pl.BlockSpec((_BN, C), lambda i: (i, 0)),
                  pl.BlockSpec((8, C), lambda i: (0, 0)),
                  pl.BlockSpec((8, C), lambda i: (0, 0))],
        out_specs=pl.BlockSpec((_BN, C), lambda i: (i, 0)),
        out_shape=jax.ShapeDtypeStruct((_NPAD, C), jnp.float32),
    )(Y, S, P)


def _inv_body(g_ref, w_ref, oh_ref, o_ref):
    acc = jnp.zeros(o_ref.shape, jnp.float32)
    g = g_ref[...]
    ohb = oh_ref[...]
    for k in range(8):
        acc = acc + ohb[:, k:k + 1] * jnp.dot(
            g, w_ref[k], preferred_element_type=jnp.float32)
    o_ref[...] = acc


def _inv_call(Ginv, Wu, oh):
    return pl.pallas_call(
        _inv_body, grid=(_NBLK,),
        in_specs=[pl.BlockSpec((_BN, 32), lambda i: (i, 0)),
                  pl.BlockSpec((8, 32, 16), lambda i: (0, 0, 0)),
                  pl.BlockSpec((_BN, 8), lambda i: (i, 0))],
        out_specs=pl.BlockSpec((_BN, 16), lambda i: (i, 0)),
        out_shape=jax.ShapeDtypeStruct((_NPAD, 16), jnp.float32),
    )(Ginv, Wu, oh)


def _final_body(y_ref, s_ref, p_ref, wo_ref, bo_ref, o_ref):
    scale, shift, cntf = _affine(s_ref, p_ref)
    y = y_ref[...]
    act = jnp.maximum(y * scale[None, :] + shift[None, :], 0.0)
    rows = (lax.broadcasted_iota(jnp.int32, y.shape, 0)
            + pl.program_id(0) * y.shape[0])
    act = jnp.where(rows < cntf.astype(jnp.int32), act, 0.0)
    o_ref[...] = (jnp.dot(act, wo_ref[...], preferred_element_type=jnp.float32)
                  + bo_ref[0:1, :])


def _final_call(Y, S, P, Wo, bo):
    bo8 = jnp.broadcast_to(bo[None, :], (8, 8))
    return pl.pallas_call(
        _final_body, grid=(_NBLK,),
        in_specs=[pl.BlockSpec((_BN, 16), lambda i: (i, 0)),
                  pl.BlockSpec((8, 16), lambda i: (0, 0)),
                  pl.BlockSpec((8, 16), lambda i: (0, 0)),
                  pl.BlockSpec((16, 8), lambda i: (0, 0)),
                  pl.BlockSpec((8, 8), lambda i: (0, 0))],
        out_specs=pl.BlockSpec((_BN, 8), lambda i: (i, 0)),
        out_shape=jax.ShapeDtypeStruct((_NPAD, 8), jnp.float32),
    )(Y, S, P, Wo, bo8)


def _params(g, b, cntf):
    C = g.shape[0]
    p = jnp.zeros((8, C), jnp.float32)
    return p.at[0].set(g).at[1].set(b).at[2].set(cntf)


def kernel(feats, coords, W1a, g1a, b1a, W1b, g1b, b1b, Wd, W2a, g2a, b2a,
           W2b, g2b, b2b, Wu, W3a, g3a, b3a, W3b, g3b, b3b, Wo, bo):
    subm1, subm2, down, inv_row, oh, cntf = _build_maps(coords)
    nf = jnp.float32(_NPT)

    feats16 = jnp.zeros((_NPAD, 16), jnp.float32).at[:_NPT, :2].set(feats)
    W1a_p = jnp.zeros((27, 16, 16), jnp.float32).at[:, :2, :].set(W1a)

    Y, S = _conv_call(_sc_gather(feats16, subm1), W1a_p, True)
    act1 = _norm_call(Y, S, _params(g1a, b1a, nf))

    Y, S = _conv_call(_sc_gather(act1, subm1), W1b, True)
    skip1 = _norm_call(Y, S, _params(g1b, b1b, nf))

    (xd,) = _conv_call(_sc_gather(skip1, down), Wd, False)

    Y, S = _conv_call(_sc_gather(xd, subm2), W2a, True)
    act2a = _norm_call(Y, S, _params(g2a, b2a, cntf))

    Y, S = _conv_call(_sc_gather(act2a, subm2), W2b, True)
    act2b = _norm_call(Y, S, _params(g2b, b2b, cntf))

    up = _inv_call(_sc_gather(act2b, inv_row)[0], Wu, oh)
    cat = jnp.concatenate([up, skip1], axis=1)

    Y, S = _conv_call(_sc_gather(cat, subm1), W3a, True)
    act3a = _norm_call(Y, S, _params(g3a, b3a, nf))

    Y, S = _conv_call(_sc_gather(act3a, subm1), W3b, True)
    return _final_call(Y, S, _params(g3b, b3b, nf), Wo, bo)[:_NPT]
